# baseline (speedup 1.0000x reference)
import functools

import jax
import jax.numpy as jnp
from jax import lax
from jax.experimental import pallas as pl
from jax.experimental.pallas import tpu as pltpu
from jax.experimental.pallas import tpu_sc as plsc

B = 16
R, C = 2048, 128
P = R * C
M = 1023
GI = 4
NW = 32
PER_TEC = B * P // NW
CROWS = 64
HB = (M + 1) * 2 * 16


def _prep_body(l_ref, t_ref, emax_ref, qq_ref):
    p = pl.program_id(1)
    l = l_ref[:, 0]
    t = t_ref[:, 0]
    lbits = lax.bitcast_convert_type(l, jnp.int32)
    ls = lax.bitcast_convert_type(lbits ^ ((t ^ 1) << 31), jnp.float32)

    @pl.when(p == 0)
    def _():
        mx = 1.0 - jnp.min(ls, axis=(1, 2))
        emax_ref[...] = jnp.broadcast_to(mx[:, None, None], (GI, 1, 128))

    @pl.when(p == 1)
    def _():
        em = emax_ref[:, 0, 0][:, None, None]
        inv = M / jnp.maximum(em, 1e-20)
        u = jnp.maximum(jnp.minimum((inv + 0.5) - ls * inv, M + 0.4), 0.0)
        q = u.astype(jnp.int32)
        lane = lax.broadcasted_iota(jnp.int32, (GI, 512, 512), 2) & 15
        idx = (q << 5) + ((t << 4) | lane)
        p0 = idx[:, 0:128, :] | (idx[:, 128:256, :] << 16)
        p1 = idx[:, 256:384, :] | (idx[:, 384:512, :] << 16)
        qq_ref[...] = jnp.concatenate([p0, p1], axis=1)


def _sc_hist_kernel(qq_hbm, out_hbm, buf0, buf1, hist, sem0, sem1):
    wid = lax.axis_index("s") * 2 + lax.axis_index("c")
    img = lax.axis_index("s")
    rbase = lax.axis_index("c") * 128
    zeros = jnp.zeros((16,), jnp.int32)
    ones = jnp.ones((16,), jnp.int32)

    @plsc.parallel_loop(0, HB, 128, unroll=2)
    def _zero(base):
        for u in range(8):
            hist[pl.ds(base + u * 16, 16)] = zeros

    nch = 128 // CROWS
    bufs = (buf0, buf1)
    sems = (sem0, sem1)
    handles = [None, None]
    handles[0] = pltpu.async_copy(
        qq_hbm.at[img, pl.ds(rbase, CROWS), :], buf0, sem0)
    for ci in range(nch):
        b = ci & 1
        if ci + 1 < nch:
            handles[1 - b] = pltpu.async_copy(
                qq_hbm.at[img, pl.ds(rbase + (ci + 1) * CROWS, CROWS), :],
                bufs[1 - b], sems[1 - b])
        handles[b].wait()
        buf = bufs[b]

        def _scatter(vi, buf=buf):
            r = vi >> 9
            c0 = vi & 511
            for u in range(16):
                w = buf[r, pl.ds(c0 + u * 16, 16)]
                plsc.addupdate_scatter(hist, [w & 0xFFFF], ones)
                plsc.addupdate_scatter(
                    hist, [lax.shift_right_logical(w, 16)], ones)

        plsc.parallel_loop(0, CROWS * 512, 256, unroll=4)(_scatter)
    pltpu.sync_copy(hist, out_hbm.at[wid])


def _finish_body(x_ref, m_ref, o_ref):
    x = x_ref[...].astype(jnp.float32)
    xr = x.reshape(B, 2, HB)
    c = xr[:, 0, :] + xr[:, 1, :]

    def shl(a, k):
        return jnp.concatenate(
            [a[:, k:], jnp.zeros((B, k), jnp.float32)], axis=1)

    for k in (1, 2, 4, 8):
        c = c + shl(c, k)
    col = lax.broadcasted_iota(jnp.int32, (B, HB), 1)
    c = jnp.where(col % 16 == 0, c, 0.0)
    k = 32
    while k <= 32 * M:
        c = c + shl(c, k)
        k *= 2
    s1 = shl(c, 16)
    m = c[:, 16:17]
    n = c[:, 0:1]
    d_pos = m + c
    g_pos = jnp.where(d_pos > 0.0, 1.0 - (m - s1) / jnp.maximum(d_pos, 0.5), 0.0)
    d_neg = n + s1
    g_neg = jnp.where(d_neg > 0.0, 1.0 - (n - c) / jnp.maximum(d_neg, 0.5), 0.0)
    valid = (col % 32 == 0) & (col >= 32)
    tot = jnp.where(valid, g_pos + g_neg, 0.0)
    per_img = jnp.sum(tot, axis=1, keepdims=True)
    delta = jnp.maximum(m_ref[:, 0, 0:1], 1e-20) * (1.0 / M)
    total = jnp.sum(per_img * delta) * (0.5 / B)
    o_ref[...] = jnp.full((1, 128), total, dtype=jnp.float32)


def kernel(logits, targets):
    l4 = logits.reshape(B, 1, 512, 512)
    t4 = targets.reshape(B, 1, 512, 512)

    emax, qq = pl.pallas_call(
        _prep_body,
        grid=(B // GI, 2),
        in_specs=[
            pl.BlockSpec((GI, 1, 512, 512), lambda i, p: (i, 0, 0, 0)),
            pl.BlockSpec((GI, 1, 512, 512), lambda i, p: (i, 0, 0, 0)),
        ],
        out_specs=[
            pl.BlockSpec((GI, 1, 128), lambda i, p: (i, 0, 0)),
            pl.BlockSpec((GI, 256, 512), lambda i, p: (i, 0, 0)),
        ],
        out_shape=[
            jax.ShapeDtypeStruct((B, 1, 128), jnp.float32),
            jax.ShapeDtypeStruct((B, 256, 512), jnp.int32),
        ],
    )(l4, t4)

    hist = functools.partial(
        pl.kernel,
        mesh=plsc.VectorSubcoreMesh(core_axis_name="c", subcore_axis_name="s"),
        out_type=jax.ShapeDtypeStruct((NW, HB), jnp.int32),
        scratch_types=[
            pltpu.VMEM((CROWS, 512), jnp.int32),
            pltpu.VMEM((CROWS, 512), jnp.int32),
            pltpu.VMEM((HB,), jnp.int32),
            pltpu.SemaphoreType.DMA,
            pltpu.SemaphoreType.DMA,
        ],
        compiler_params=pltpu.CompilerParams(needs_layout_passes=False),
    )(_sc_hist_kernel)(qq)

    out = pl.pallas_call(
        _finish_body,
        grid=(1,),
        in_specs=[
            pl.BlockSpec((NW, HB), lambda i: (0, 0)),
            pl.BlockSpec((B, 1, 128), lambda i: (0, 0, 0)),
        ],
        out_specs=pl.BlockSpec((1, 128), lambda i: (0, 0)),
        out_shape=jax.ShapeDtypeStruct((1, 128), jnp.float32),
    )(hist, emax)

    return out[0, 0]


# speedup vs baseline: 1.0151x; 1.0151x over previous
import functools

import jax
import jax.numpy as jnp
from jax import lax
from jax.experimental import pallas as pl
from jax.experimental.pallas import tpu as pltpu
from jax.experimental.pallas import tpu_sc as plsc

B = 16
R, C = 2048, 128
P = R * C
M = 1023
GI = 4
NW = 32
PER_TEC = B * P // NW
CROWS = 32
HB = (M + 1) * 2 * 16


def _prep_body(l_ref, t_ref, emax_ref, qq_ref):
    p = pl.program_id(1)
    l = l_ref[:, 0]
    t = t_ref[:, 0]
    lbits = lax.bitcast_convert_type(l, jnp.int32)
    ls = lax.bitcast_convert_type(lbits ^ ((t ^ 1) << 31), jnp.float32)

    @pl.when(p == 0)
    def _():
        mx = 1.0 - jnp.min(ls, axis=(1, 2))
        emax_ref[...] = jnp.broadcast_to(mx[:, None, None], (GI, 1, 128))

    @pl.when(p == 1)
    def _():
        em = emax_ref[:, 0, 0][:, None, None]
        inv = M / jnp.maximum(em, 1e-20)
        u = jnp.maximum(jnp.minimum((inv + 0.5) - ls * inv, M + 0.4), 0.0)
        q = u.astype(jnp.int32)
        lane = lax.broadcasted_iota(jnp.int32, (GI, 512, 512), 2) & 15
        idx = (q << 5) + ((t << 4) | lane)
        p0 = idx[:, 0:128, :] | (idx[:, 128:256, :] << 16)
        p1 = idx[:, 256:384, :] | (idx[:, 384:512, :] << 16)
        qq_ref[...] = jnp.concatenate([p0, p1], axis=1)


def _sc_hist_kernel(qq_hbm, out_hbm, buf0, buf1, hist, sem0, sem1):
    wid = lax.axis_index("s") * 2 + lax.axis_index("c")
    img = lax.axis_index("s")
    rbase = lax.axis_index("c") * 128
    zeros = jnp.zeros((16,), jnp.int32)
    ones = jnp.ones((16,), jnp.int32)

    @plsc.parallel_loop(0, HB, 128, unroll=2)
    def _zero(base):
        for u in range(8):
            hist[pl.ds(base + u * 16, 16)] = zeros

    nch = 128 // CROWS
    bufs = (buf0, buf1)
    sems = (sem0, sem1)
    handles = [None, None]
    handles[0] = pltpu.async_copy(
        qq_hbm.at[img, pl.ds(rbase, CROWS), :], buf0, sem0)
    for ci in range(nch):
        b = ci & 1
        if ci + 1 < nch:
            handles[1 - b] = pltpu.async_copy(
                qq_hbm.at[img, pl.ds(rbase + (ci + 1) * CROWS, CROWS), :],
                bufs[1 - b], sems[1 - b])
        handles[b].wait()
        buf = bufs[b]

        def _scatter(vi, buf=buf):
            r = vi >> 9
            c0 = vi & 511
            for u in range(16):
                w = buf[r, pl.ds(c0 + u * 16, 16)]
                plsc.addupdate_scatter(hist, [w & 0xFFFF], ones)
                plsc.addupdate_scatter(
                    hist, [lax.shift_right_logical(w, 16)], ones)

        plsc.parallel_loop(0, CROWS * 512, 256, unroll=2)(_scatter)
    pltpu.sync_copy(hist, out_hbm.at[wid])


def _finish_body(x_ref, m_ref, o_ref):
    x = x_ref[...].astype(jnp.float32)
    xr = x.reshape(B, 2, HB)
    c = xr[:, 0, :] + xr[:, 1, :]

    def shl(a, k):
        return jnp.concatenate(
            [a[:, k:], jnp.zeros((B, k), jnp.float32)], axis=1)

    for k in (1, 2, 4, 8):
        c = c + shl(c, k)
    col = lax.broadcasted_iota(jnp.int32, (B, HB), 1)
    c = jnp.where(col % 16 == 0, c, 0.0)
    k = 32
    while k <= 32 * M:
        c = c + shl(c, k)
        k *= 2
    s1 = shl(c, 16)
    m = c[:, 16:17]
    n = c[:, 0:1]
    d_pos = m + c
    g_pos = jnp.where(d_pos > 0.0, 1.0 - (m - s1) / jnp.maximum(d_pos, 0.5), 0.0)
    d_neg = n + s1
    g_neg = jnp.where(d_neg > 0.0, 1.0 - (n - c) / jnp.maximum(d_neg, 0.5), 0.0)
    valid = (col % 32 == 0) & (col >= 32)
    tot = jnp.where(valid, g_pos + g_neg, 0.0)
    per_img = jnp.sum(tot, axis=1, keepdims=True)
    delta = jnp.maximum(m_ref[:, 0, 0:1], 1e-20) * (1.0 / M)
    total = jnp.sum(per_img * delta) * (0.5 / B)
    o_ref[...] = jnp.full((1, 128), total, dtype=jnp.float32)


def kernel(logits, targets):
    l4 = logits.reshape(B, 1, 512, 512)
    t4 = targets.reshape(B, 1, 512, 512)

    emax, qq = pl.pallas_call(
        _prep_body,
        grid=(B // GI, 2),
        in_specs=[
            pl.BlockSpec((GI, 1, 512, 512), lambda i, p: (i, 0, 0, 0)),
            pl.BlockSpec((GI, 1, 512, 512), lambda i, p: (i, 0, 0, 0)),
        ],
        out_specs=[
            pl.BlockSpec((GI, 1, 128), lambda i, p: (i, 0, 0)),
            pl.BlockSpec((GI, 256, 512), lambda i, p: (i, 0, 0)),
        ],
        out_shape=[
            jax.ShapeDtypeStruct((B, 1, 128), jnp.float32),
            jax.ShapeDtypeStruct((B, 256, 512), jnp.int32),
        ],
    )(l4, t4)

    hist = functools.partial(
        pl.kernel,
        mesh=plsc.VectorSubcoreMesh(core_axis_name="c", subcore_axis_name="s"),
        out_type=jax.ShapeDtypeStruct((NW, HB), jnp.int32),
        scratch_types=[
            pltpu.VMEM((CROWS, 512), jnp.int32),
            pltpu.VMEM((CROWS, 512), jnp.int32),
            pltpu.VMEM((HB,), jnp.int32),
            pltpu.SemaphoreType.DMA,
            pltpu.SemaphoreType.DMA,
        ],
        compiler_params=pltpu.CompilerParams(needs_layout_passes=False),
    )(_sc_hist_kernel)(qq)

    out = pl.pallas_call(
        _finish_body,
        grid=(1,),
        in_specs=[
            pl.BlockSpec((NW, HB), lambda i: (0, 0)),
            pl.BlockSpec((B, 1, 128), lambda i: (0, 0, 0)),
        ],
        out_specs=pl.BlockSpec((1, 128), lambda i: (0, 0)),
        out_shape=jax.ShapeDtypeStruct((1, 128), jnp.float32),
    )(hist, emax)

    return out[0, 0]
